# Initial kernel scaffold; baseline (speedup 1.0000x reference)
import jax
import jax.numpy as jnp
from jax.experimental import pallas as pl


def kernel(x, edge_index, edge_attr, W_np, b_np, W_ep, b_ep, W_nt, b_nt, W_et, b_et):
    raise NotImplementedError("write your pallas kernel here")


# speedup vs baseline: 2.3458x; 2.3458x over previous
import functools

import jax
import jax.numpy as jnp
from jax import lax
from jax.experimental import pallas as pl
from jax.experimental.pallas import tpu as pltpu
from jax.experimental.pallas import tpu_sc as plsc

N = 10000
E = 320000
D = 128
DE = 16
DH = 128
DEH = 16

NC = 2
NS = 16
LANES = 16

N_PAD = 10240
DUMMY = N
E_PAD = 323584
BLK = 128
CHUNK = E_PAD // NS
NBLK = CHUNK // BLK
EO_CHUNK = E_PAD // (NC * NS)
EO_NBLK = EO_CHUNK // BLK
ROWS_PT = N_PAD // NS


def _tc_node_pre_body(x_ref, wnp_ref, bnp_ref, we0_ref, we2_ref,
                      h_ref, hs_ref, hd_ref):
    hb = jnp.dot(x_ref[...], wnp_ref[...],
                 preferred_element_type=jnp.float32) + bnp_ref[...]
    h_ref[...] = hb
    hs_ref[...] = jnp.dot(hb, we0_ref[...], preferred_element_type=jnp.float32)
    hd_ref[...] = jnp.dot(hb, we2_ref[...], preferred_element_type=jnp.float32)


def _tc_node_pre(x_pad, W_np, b_np, We0, We2):
    nb = N_PAD // 1024
    return pl.pallas_call(
        _tc_node_pre_body,
        grid=(nb,),
        in_specs=[
            pl.BlockSpec((1024, D), lambda i: (i, 0)),
            pl.BlockSpec((D, DH), lambda i: (0, 0)),
            pl.BlockSpec((DH,), lambda i: (0,)),
            pl.BlockSpec((DH, DE), lambda i: (0, 0)),
            pl.BlockSpec((DH, DE), lambda i: (0, 0)),
        ],
        out_specs=[
            pl.BlockSpec((1024, DH), lambda i: (i, 0)),
            pl.BlockSpec((1024, DE), lambda i: (i, 0)),
            pl.BlockSpec((1024, DE), lambda i: (i, 0)),
        ],
        out_shape=[
            jax.ShapeDtypeStruct((N_PAD, DH), jnp.float32),
            jax.ShapeDtypeStruct((N_PAD, DE), jnp.float32),
            jax.ShapeDtypeStruct((N_PAD, DE), jnp.float32),
        ],
    )(x_pad, W_np, b_np, We0, We2)


def _tc_edge_pre_body(ea_ref, wep_ref, bep_ref, we1_ref, bet_ref,
                      e_ref, ee_ref):
    eb = jnp.dot(ea_ref[...], wep_ref[...],
                 preferred_element_type=jnp.float32) + bep_ref[...]
    e_ref[...] = eb
    ee_ref[...] = jnp.dot(eb, we1_ref[...],
                          preferred_element_type=jnp.float32) + bet_ref[...]


def _tc_edge_pre(edge_attr, W_ep, b_ep, We1, b_et):
    eblk = 4000
    nb = E // eblk
    return pl.pallas_call(
        _tc_edge_pre_body,
        grid=(nb,),
        in_specs=[
            pl.BlockSpec((eblk, DE), lambda i: (i, 0)),
            pl.BlockSpec((DE, DEH), lambda i: (0, 0)),
            pl.BlockSpec((DEH,), lambda i: (0,)),
            pl.BlockSpec((DEH, DE), lambda i: (0, 0)),
            pl.BlockSpec((DE,), lambda i: (0,)),
        ],
        out_specs=[
            pl.BlockSpec((eblk, DEH), lambda i: (i, 0)),
            pl.BlockSpec((eblk, DE), lambda i: (i, 0)),
        ],
        out_shape=[
            jax.ShapeDtypeStruct((E, DEH), jnp.float32),
            jax.ShapeDtypeStruct((E, DE), jnp.float32),
        ],
    )(edge_attr, W_ep, b_ep, We1, b_et)


_ZERO16 = functools.partial(jnp.zeros, (LANES,), jnp.float32)


def _sc_graph_body(h_hbm, e_hbm, gidx_hbm, sidx_hbm, hs_hbm, hd_hbm, ee_hbm,
                   shi_hbm, sho_hbm, sei_hbm, seo_hbm, eo_hbm, deg_sh,
                   acc_h, acc_e,
                   rows_v, erow_v, gi_v, si_v, hist_v, degr_v, tmp_v,
                   st_h, st_e, a_v, b_v, c_v, sem, sem2):
    c = lax.axis_index("c")
    s = lax.axis_index("s")

    def zero_rows(i, _):
        for k in range(DH // LANES):
            rows_v[i, pl.ds(k * LANES, LANES)] = _ZERO16()
        erow_v[i] = _ZERO16()
        return ()

    lax.fori_loop(0, BLK, zero_rows, ())

    def zero_acc(g, _):
        r = s * ROWS_PT + g * BLK
        pltpu.sync_copy(rows_v, acc_h.at[pl.ds(r, BLK)])
        pltpu.sync_copy(erow_v, acc_e.at[pl.ds(r, BLK)])
        return ()

    lax.fori_loop(0, ROWS_PT // BLK, zero_acc, ())

    def zero_hist(i, _):
        hist_v[pl.ds(i * LANES, LANES)] = _ZERO16()
        return ()

    lax.fori_loop(0, N_PAD // LANES, zero_hist, ())
    plsc.subcore_barrier()

    base = s * CHUNK

    def blk_body(b, _):
        off = base + b * BLK
        pltpu.sync_copy(gidx_hbm.at[c, pl.ds(off, BLK)], gi_v)
        pltpu.sync_copy(sidx_hbm.at[c, pl.ds(off, BLK)], si_v.at[0])
        cp = pltpu.async_copy(h_hbm.at[gi_v], rows_v, sem)
        pltpu.sync_copy(e_hbm.at[pl.ds(off, BLK)], erow_v)
        for j in range(BLK // LANES):
            iv = si_v[0, pl.ds(j * LANES, LANES)]
            plsc.addupdate_scatter(hist_v, [iv],
                                   jnp.ones((LANES,), jnp.float32))
        cp.wait()
        pltpu.sync_copy(rows_v, acc_h.at[si_v.at[0]], add=True)
        pltpu.sync_copy(erow_v, acc_e.at[si_v.at[0]], add=True)
        return ()

    lax.fori_loop(0, NBLK, blk_body, ())

    pltpu.sync_copy(hist_v, deg_sh.at[c, s])
    plsc.subcore_barrier()

    r0 = s * ROWS_PT

    def red_init(i, _):
        degr_v[pl.ds(i * LANES, LANES)] = _ZERO16()
        return ()

    lax.fori_loop(0, ROWS_PT // LANES, red_init, ())

    def red_j(j, _):
        pltpu.sync_copy(deg_sh.at[c, j, pl.ds(r0, ROWS_PT)], tmp_v)

        def addk(k, _):
            sl = pl.ds(k * LANES, LANES)
            degr_v[sl] = degr_v[sl] + tmp_v[sl]
            return ()

        lax.fori_loop(0, ROWS_PT // LANES, addk, ())
        return ()

    lax.fori_loop(0, NS, red_j, ())

    def scale_g(g, _):
        rr = r0 + g * LANES
        pltpu.sync_copy(acc_h.at[pl.ds(rr, LANES)], st_h)
        pltpu.sync_copy(acc_e.at[pl.ds(rr, LANES)], st_e)
        deg16 = degr_v[pl.ds(g * LANES, LANES)]
        r16 = 1.0 / jnp.maximum(deg16, 1.0)
        for i in range(LANES):
            r_s = r16[i]
            for k in range(DH // LANES):
                sl = pl.ds(k * LANES, LANES)
                st_h[i, sl] = st_h[i, sl] * r_s
            st_e[i] = st_e[i] * r_s

        @pl.when(c == 0)
        def _():
            pltpu.sync_copy(st_h, shi_hbm.at[pl.ds(rr, LANES)])
            pltpu.sync_copy(st_e, sei_hbm.at[pl.ds(rr, LANES)])

        @pl.when(c == 1)
        def _():
            pltpu.sync_copy(st_h, sho_hbm.at[pl.ds(rr, LANES)])
            pltpu.sync_copy(st_e, seo_hbm.at[pl.ds(rr, LANES)])

        return ()

    lax.fori_loop(0, ROWS_PT // LANES, scale_g, ())

    eo_base = (s * NC + c) * EO_CHUNK

    def eo_body(b, _):
        off = eo_base + b * BLK
        pltpu.sync_copy(gidx_hbm.at[0, pl.ds(off, BLK)], gi_v)
        pltpu.sync_copy(gidx_hbm.at[1, pl.ds(off, BLK)], si_v.at[0])
        cp1 = pltpu.async_copy(hs_hbm.at[gi_v], a_v, sem)
        cp2 = pltpu.async_copy(hd_hbm.at[si_v.at[0]], b_v, sem2)
        pltpu.sync_copy(ee_hbm.at[pl.ds(off, BLK)], c_v)
        cp1.wait()
        cp2.wait()

        def rowadd(i, _):
            a_v[i] = a_v[i] + b_v[i] + c_v[i]
            return ()

        lax.fori_loop(0, BLK, rowadd, ())
        pltpu.sync_copy(a_v, eo_hbm.at[pl.ds(off, BLK)])
        return ()

    lax.fori_loop(0, EO_NBLK, eo_body, ())


_sc_graph = functools.partial(
    pl.kernel,
    out_type=[
        jax.ShapeDtypeStruct((N_PAD, DH), jnp.float32),
        jax.ShapeDtypeStruct((N_PAD, DH), jnp.float32),
        jax.ShapeDtypeStruct((N_PAD, DEH), jnp.float32),
        jax.ShapeDtypeStruct((N_PAD, DEH), jnp.float32),
        jax.ShapeDtypeStruct((E_PAD, DE), jnp.float32),
        jax.ShapeDtypeStruct((NC, NS, N_PAD), jnp.float32),
    ],
    mesh=plsc.VectorSubcoreMesh(core_axis_name="c", subcore_axis_name="s"),
    compiler_params=pltpu.CompilerParams(
        needs_layout_passes=False, use_tc_tiling_on_sc=False),
    scratch_types=[
        pltpu.VMEM_SHARED((N_PAD, DH), jnp.float32),
        pltpu.VMEM_SHARED((N_PAD, DEH), jnp.float32),
        pltpu.VMEM((BLK, DH), jnp.float32),
        pltpu.VMEM((BLK, DEH), jnp.float32),
        pltpu.VMEM((BLK,), jnp.int32),
        pltpu.VMEM((1, BLK), jnp.int32),
        pltpu.VMEM((N_PAD,), jnp.float32),
        pltpu.VMEM((ROWS_PT,), jnp.float32),
        pltpu.VMEM((ROWS_PT,), jnp.float32),
        pltpu.VMEM((LANES, DH), jnp.float32),
        pltpu.VMEM((LANES, DEH), jnp.float32),
        pltpu.VMEM((BLK, DE), jnp.float32),
        pltpu.VMEM((BLK, DE), jnp.float32),
        pltpu.VMEM((BLK, DE), jnp.float32),
        pltpu.SemaphoreType.DMA,
        pltpu.SemaphoreType.DMA,
    ],
)(_sc_graph_body)


def _tc_node_post_body(shi_ref, sei_ref, h_ref, sho_ref, seo_ref,
                       w1_ref, w2_ref, w3_ref, w4_ref, w5_ref, bnt_ref,
                       out_ref):
    acc = jnp.dot(shi_ref[...], w1_ref[...], preferred_element_type=jnp.float32)
    acc = acc + jnp.dot(sei_ref[...], w2_ref[...],
                        preferred_element_type=jnp.float32)
    acc = acc + jnp.dot(h_ref[...], w3_ref[...],
                        preferred_element_type=jnp.float32)
    acc = acc + jnp.dot(sho_ref[...], w4_ref[...],
                        preferred_element_type=jnp.float32)
    acc = acc + jnp.dot(seo_ref[...], w5_ref[...],
                        preferred_element_type=jnp.float32)
    out_ref[...] = acc + bnt_ref[...]


def _tc_node_post(shi, sei, h_pad, sho, seo, W1, W2, W3, W4, W5, b_nt):
    nb = N_PAD // 1024
    return pl.pallas_call(
        _tc_node_post_body,
        grid=(nb,),
        in_specs=[
            pl.BlockSpec((1024, DH), lambda i: (i, 0)),
            pl.BlockSpec((1024, DEH), lambda i: (i, 0)),
            pl.BlockSpec((1024, DH), lambda i: (i, 0)),
            pl.BlockSpec((1024, DH), lambda i: (i, 0)),
            pl.BlockSpec((1024, DEH), lambda i: (i, 0)),
            pl.BlockSpec((DH, D), lambda i: (0, 0)),
            pl.BlockSpec((DEH, D), lambda i: (0, 0)),
            pl.BlockSpec((DH, D), lambda i: (0, 0)),
            pl.BlockSpec((DH, D), lambda i: (0, 0)),
            pl.BlockSpec((DEH, D), lambda i: (0, 0)),
            pl.BlockSpec((D,), lambda i: (0,)),
        ],
        out_specs=pl.BlockSpec((1024, D), lambda i: (i, 0)),
        out_shape=jax.ShapeDtypeStruct((N_PAD, D), jnp.float32),
    )(shi, sei, h_pad, sho, seo, W1, W2, W3, W4, W5, b_nt)


def kernel(x, edge_index, edge_attr, W_np, b_np, W_ep, b_ep,
           W_nt, b_nt, W_et, b_et):
    src = edge_index[0]
    dst = edge_index[1]
    pad = E_PAD - E
    zpad = jnp.zeros((pad,), jnp.int32)
    dpad = jnp.full((pad,), DUMMY, jnp.int32)
    gidx = jnp.stack([jnp.concatenate([src, zpad]),
                      jnp.concatenate([dst, zpad])])
    sidx = jnp.stack([jnp.concatenate([dst, dpad]),
                      jnp.concatenate([src, dpad])])

    x_pad = jnp.concatenate(
        [x, jnp.zeros((N_PAD - N, D), jnp.float32)], axis=0)

    h_pad, hs_et, hd_et = _tc_node_pre(
        x_pad, W_np, b_np, W_et[0:DH], W_et[DH + DEH:])
    e, ee = _tc_edge_pre(edge_attr, W_ep, b_ep, W_et[DH:DH + DEH], b_et)
    e_pad = jnp.concatenate(
        [e, jnp.zeros((pad, DEH), jnp.float32)], axis=0)
    ee_pad = jnp.concatenate(
        [ee, jnp.zeros((pad, DE), jnp.float32)], axis=0)

    shi, sho, sei, seo, eo, _ = _sc_graph(
        h_pad, e_pad, gidx, sidx, hs_et, hd_et, ee_pad)

    node_out = _tc_node_post(
        shi, sei, h_pad, sho, seo,
        W_nt[0:DH], W_nt[DH:DH + DEH], W_nt[DH + DEH:2 * DH + DEH],
        W_nt[2 * DH + DEH:3 * DH + DEH], W_nt[3 * DH + DEH:], b_nt)

    return node_out[:N], eo[:E]


# speedup vs baseline: 3.2510x; 1.3859x over previous
import functools

import jax
import jax.numpy as jnp
from jax import lax
from jax.experimental import pallas as pl
from jax.experimental.pallas import tpu as pltpu
from jax.experimental.pallas import tpu_sc as plsc

N = 10000
E = 320000
D = 128
DE = 16
DH = 128
DEH = 16

NC = 2
NS = 16
LANES = 16

N_PAD = 10240
BLK = 128
NBLK_ALL = E // BLK
ROWS_PT = N_PAD // NS


def _tc_node_pre_body(x_ref, wnp_ref, bnp_ref, we0_ref, we2_ref,
                      h_ref, hs_ref, hd_ref):
    hb = jnp.dot(x_ref[...], wnp_ref[...],
                 preferred_element_type=jnp.float32) + bnp_ref[...]
    h_ref[...] = hb
    hs_ref[...] = jnp.dot(hb, we0_ref[...], preferred_element_type=jnp.float32)
    hd_ref[...] = jnp.dot(hb, we2_ref[...], preferred_element_type=jnp.float32)


def _tc_node_pre(x, W_np, b_np, We0, We2):
    blk = 1000
    return pl.pallas_call(
        _tc_node_pre_body,
        grid=(N // blk,),
        in_specs=[
            pl.BlockSpec((blk, D), lambda i: (i, 0)),
            pl.BlockSpec((D, DH), lambda i: (0, 0)),
            pl.BlockSpec((DH,), lambda i: (0,)),
            pl.BlockSpec((DH, DE), lambda i: (0, 0)),
            pl.BlockSpec((DH, DE), lambda i: (0, 0)),
        ],
        out_specs=[
            pl.BlockSpec((blk, DH), lambda i: (i, 0)),
            pl.BlockSpec((blk, DE), lambda i: (i, 0)),
            pl.BlockSpec((blk, DE), lambda i: (i, 0)),
        ],
        out_shape=[
            jax.ShapeDtypeStruct((N, DH), jnp.float32),
            jax.ShapeDtypeStruct((N, DE), jnp.float32),
            jax.ShapeDtypeStruct((N, DE), jnp.float32),
        ],
    )(x, W_np, b_np, We0, We2)


def _tc_edge_pre_body(ea_ref, wep_ref, bep_ref, we1_ref, bet_ref,
                      e_ref, ee_ref):
    eb = jnp.dot(ea_ref[...], wep_ref[...],
                 preferred_element_type=jnp.float32) + bep_ref[...]
    e_ref[...] = eb
    ee_ref[...] = jnp.dot(eb, we1_ref[...],
                          preferred_element_type=jnp.float32) + bet_ref[...]


def _tc_edge_pre(edge_attr, W_ep, b_ep, We1, b_et):
    eblk = 8000
    return pl.pallas_call(
        _tc_edge_pre_body,
        grid=(E // eblk,),
        in_specs=[
            pl.BlockSpec((eblk, DE), lambda i: (i, 0)),
            pl.BlockSpec((DE, DEH), lambda i: (0, 0)),
            pl.BlockSpec((DEH,), lambda i: (0,)),
            pl.BlockSpec((DEH, DE), lambda i: (0, 0)),
            pl.BlockSpec((DE,), lambda i: (0,)),
        ],
        out_specs=[
            pl.BlockSpec((eblk, DEH), lambda i: (i, 0)),
            pl.BlockSpec((eblk, DE), lambda i: (i, 0)),
        ],
        out_shape=[
            jax.ShapeDtypeStruct((E, DEH), jnp.float32),
            jax.ShapeDtypeStruct((E, DE), jnp.float32),
        ],
    )(edge_attr, W_ep, b_ep, We1, b_et)


_ZERO16 = functools.partial(jnp.zeros, (LANES,), jnp.float32)


def _sc_graph_body(h_hbm, e_hbm, src_hbm, dst_hbm, hs_hbm, hd_hbm, ee_hbm,
                   shi_hbm, sho_hbm, sei_hbm, seo_hbm, eo_hbm, deg_sh,
                   acc_h, acc_e,
                   rows_v, erow_v, gi_v, si_v, hist_v, degr_v, tmp_v,
                   st_h, st_e, a_v, b_v, c_v, sem, sem2):
    c = lax.axis_index("c")
    s = lax.axis_index("s")

    def zero_rows(i, _):
        for k in range(DH // LANES):
            rows_v[i, pl.ds(k * LANES, LANES)] = _ZERO16()
        erow_v[i] = _ZERO16()
        return ()

    lax.fori_loop(0, BLK, zero_rows, ())

    def zero_acc(g, _):
        r = s * ROWS_PT + g * BLK
        pltpu.sync_copy(rows_v, acc_h.at[pl.ds(r, BLK)])
        pltpu.sync_copy(erow_v, acc_e.at[pl.ds(r, BLK)])
        return ()

    lax.fori_loop(0, ROWS_PT // BLK, zero_acc, ())

    def zero_hist(i, _):
        hist_v[pl.ds(i * LANES, LANES)] = _ZERO16()
        return ()

    lax.fori_loop(0, N_PAD // LANES, zero_hist, ())
    plsc.subcore_barrier()

    my_nblk = jnp.where(s < NBLK_ALL - (NBLK_ALL // NS) * NS,
                        NBLK_ALL // NS + 1, NBLK_ALL // NS)

    def run_direction(gref, sref):
        def blk_body(b, _):
            off = (b * NS + s) * BLK
            pltpu.sync_copy(gref.at[pl.ds(off, BLK)], gi_v)
            pltpu.sync_copy(sref.at[pl.ds(off, BLK)], si_v.at[0])
            cp = pltpu.async_copy(h_hbm.at[gi_v], rows_v, sem)
            pltpu.sync_copy(e_hbm.at[pl.ds(off, BLK)], erow_v)
            for j in range(BLK // LANES):
                iv = si_v[0, pl.ds(j * LANES, LANES)]
                plsc.addupdate_scatter(hist_v, [iv],
                                       jnp.ones((LANES,), jnp.float32))
            cp.wait()
            pltpu.sync_copy(rows_v, acc_h.at[si_v.at[0]], add=True)
            pltpu.sync_copy(erow_v, acc_e.at[si_v.at[0]], add=True)
            return ()

        lax.fori_loop(0, my_nblk, blk_body, ())

    @pl.when(c == 0)
    def _():
        run_direction(src_hbm, dst_hbm)

    @pl.when(c == 1)
    def _():
        run_direction(dst_hbm, src_hbm)

    pltpu.sync_copy(hist_v, deg_sh.at[c, s])
    plsc.subcore_barrier()

    r0 = s * ROWS_PT

    def red_init(i, _):
        degr_v[pl.ds(i * LANES, LANES)] = _ZERO16()
        return ()

    lax.fori_loop(0, ROWS_PT // LANES, red_init, ())

    def red_j(j, _):
        pltpu.sync_copy(deg_sh.at[c, j, pl.ds(r0, ROWS_PT)], tmp_v)

        def addk(k, _):
            sl = pl.ds(k * LANES, LANES)
            degr_v[sl] = degr_v[sl] + tmp_v[sl]
            return ()

        lax.fori_loop(0, ROWS_PT // LANES, addk, ())
        return ()

    lax.fori_loop(0, NS, red_j, ())

    def scale_g(g, _):
        rr = r0 + g * LANES
        pltpu.sync_copy(acc_h.at[pl.ds(rr, LANES)], st_h)
        pltpu.sync_copy(acc_e.at[pl.ds(rr, LANES)], st_e)
        deg16 = degr_v[pl.ds(g * LANES, LANES)]
        r16 = 1.0 / jnp.maximum(deg16, 1.0)
        for i in range(LANES):
            r_s = r16[i]
            for k in range(DH // LANES):
                sl = pl.ds(k * LANES, LANES)
                st_h[i, sl] = st_h[i, sl] * r_s
            st_e[i] = st_e[i] * r_s

        @pl.when(c == 0)
        def _():
            pltpu.sync_copy(st_h, shi_hbm.at[pl.ds(rr, LANES)])
            pltpu.sync_copy(st_e, sei_hbm.at[pl.ds(rr, LANES)])

        @pl.when(c == 1)
        def _():
            pltpu.sync_copy(st_h, sho_hbm.at[pl.ds(rr, LANES)])
            pltpu.sync_copy(st_e, seo_hbm.at[pl.ds(rr, LANES)])

        return ()

    lax.fori_loop(0, ROWS_PT // LANES, scale_g, ())

    wid = s * NC + c
    NW = NC * NS
    my_eo_nblk = jnp.where(wid < NBLK_ALL - (NBLK_ALL // NW) * NW,
                           NBLK_ALL // NW + 1, NBLK_ALL // NW)

    def eo_body(b, _):
        off = (b * NW + wid) * BLK
        pltpu.sync_copy(src_hbm.at[pl.ds(off, BLK)], gi_v)
        pltpu.sync_copy(dst_hbm.at[pl.ds(off, BLK)], si_v.at[0])
        cp1 = pltpu.async_copy(hs_hbm.at[gi_v], a_v, sem)
        cp2 = pltpu.async_copy(hd_hbm.at[si_v.at[0]], b_v, sem2)
        pltpu.sync_copy(ee_hbm.at[pl.ds(off, BLK)], c_v)
        cp1.wait()
        cp2.wait()

        def rowadd(i, _):
            a_v[i] = a_v[i] + b_v[i] + c_v[i]
            return ()

        lax.fori_loop(0, BLK, rowadd, ())
        pltpu.sync_copy(a_v, eo_hbm.at[pl.ds(off, BLK)])
        return ()

    lax.fori_loop(0, my_eo_nblk, eo_body, ())


_sc_graph = functools.partial(
    pl.kernel,
    out_type=[
        jax.ShapeDtypeStruct((N_PAD, DH), jnp.float32),
        jax.ShapeDtypeStruct((N_PAD, DH), jnp.float32),
        jax.ShapeDtypeStruct((N_PAD, DEH), jnp.float32),
        jax.ShapeDtypeStruct((N_PAD, DEH), jnp.float32),
        jax.ShapeDtypeStruct((E, DE), jnp.float32),
        jax.ShapeDtypeStruct((NC, NS, N_PAD), jnp.float32),
    ],
    mesh=plsc.VectorSubcoreMesh(core_axis_name="c", subcore_axis_name="s"),
    compiler_params=pltpu.CompilerParams(
        needs_layout_passes=False, use_tc_tiling_on_sc=False),
    scratch_types=[
        pltpu.VMEM_SHARED((N_PAD, DH), jnp.float32),
        pltpu.VMEM_SHARED((N_PAD, DEH), jnp.float32),
        pltpu.VMEM((BLK, DH), jnp.float32),
        pltpu.VMEM((BLK, DEH), jnp.float32),
        pltpu.VMEM((BLK,), jnp.int32),
        pltpu.VMEM((1, BLK), jnp.int32),
        pltpu.VMEM((N_PAD,), jnp.float32),
        pltpu.VMEM((ROWS_PT,), jnp.float32),
        pltpu.VMEM((ROWS_PT,), jnp.float32),
        pltpu.VMEM((LANES, DH), jnp.float32),
        pltpu.VMEM((LANES, DEH), jnp.float32),
        pltpu.VMEM((BLK, DE), jnp.float32),
        pltpu.VMEM((BLK, DE), jnp.float32),
        pltpu.VMEM((BLK, DE), jnp.float32),
        pltpu.SemaphoreType.DMA,
        pltpu.SemaphoreType.DMA,
    ],
)(_sc_graph_body)


def _tc_node_post_body(shi_ref, sei_ref, h_ref, sho_ref, seo_ref,
                       w1_ref, w2_ref, w3_ref, w4_ref, w5_ref, bnt_ref,
                       out_ref):
    acc = jnp.dot(shi_ref[...], w1_ref[...], preferred_element_type=jnp.float32)
    acc = acc + jnp.dot(sei_ref[...], w2_ref[...],
                        preferred_element_type=jnp.float32)
    acc = acc + jnp.dot(h_ref[...], w3_ref[...],
                        preferred_element_type=jnp.float32)
    acc = acc + jnp.dot(sho_ref[...], w4_ref[...],
                        preferred_element_type=jnp.float32)
    acc = acc + jnp.dot(seo_ref[...], w5_ref[...],
                        preferred_element_type=jnp.float32)
    out_ref[...] = acc + bnt_ref[...]


def _tc_node_post(shi, sei, h, sho, seo, W1, W2, W3, W4, W5, b_nt):
    blk = 1000
    return pl.pallas_call(
        _tc_node_post_body,
        grid=(N // blk,),
        in_specs=[
            pl.BlockSpec((blk, DH), lambda i: (i, 0)),
            pl.BlockSpec((blk, DEH), lambda i: (i, 0)),
            pl.BlockSpec((blk, DH), lambda i: (i, 0)),
            pl.BlockSpec((blk, DH), lambda i: (i, 0)),
            pl.BlockSpec((blk, DEH), lambda i: (i, 0)),
            pl.BlockSpec((DH, D), lambda i: (0, 0)),
            pl.BlockSpec((DEH, D), lambda i: (0, 0)),
            pl.BlockSpec((DH, D), lambda i: (0, 0)),
            pl.BlockSpec((DH, D), lambda i: (0, 0)),
            pl.BlockSpec((DEH, D), lambda i: (0, 0)),
            pl.BlockSpec((D,), lambda i: (0,)),
        ],
        out_specs=pl.BlockSpec((blk, D), lambda i: (i, 0)),
        out_shape=jax.ShapeDtypeStruct((N, D), jnp.float32),
    )(shi, sei, h, sho, seo, W1, W2, W3, W4, W5, b_nt)


def kernel(x, edge_index, edge_attr, W_np, b_np, W_ep, b_ep,
           W_nt, b_nt, W_et, b_et):
    src = edge_index[0]
    dst = edge_index[1]

    h, hs_et, hd_et = _tc_node_pre(
        x, W_np, b_np, W_et[0:DH], W_et[DH + DEH:])
    e, ee = _tc_edge_pre(edge_attr, W_ep, b_ep, W_et[DH:DH + DEH], b_et)

    shi, sho, sei, seo, eo, _ = _sc_graph(
        h, e, src, dst, hs_et, hd_et, ee)

    node_out = _tc_node_post(
        shi, sei, h, sho, seo,
        W_nt[0:DH], W_nt[DH:DH + DEH], W_nt[DH + DEH:2 * DH + DEH],
        W_nt[2 * DH + DEH:3 * DH + DEH], W_nt[3 * DH + DEH:], b_nt)

    return node_out, eo
